# baseline (speedup 1.0000x reference)
import functools

import jax
import jax.numpy as jnp
from jax import lax
from jax.experimental import pallas as pl
from jax.experimental.pallas import tpu as pltpu
from jax.experimental.pallas import tpu_sc as plsc

BATCH = 16384
NUM_FIELDS = 4
ATTR_DIM = 64
VOCAB = 1000

_info = plsc.get_sparse_core_info()
NC, NS, L = _info.num_cores, _info.num_subcores, _info.num_lanes
NW = NC * NS
BW = BATCH // NW
CHUNK = 512
NCHUNK = BW // CHUNK
NSTEP = NUM_FIELDS * NCHUNK
NBUF = 3

_mesh = plsc.VectorSubcoreMesh(core_axis_name="c", subcore_axis_name="s")


@functools.partial(
    pl.kernel,
    mesh=_mesh,
    compiler_params=pltpu.CompilerParams(use_tc_tiling_on_sc=False),
    out_type=jax.ShapeDtypeStruct((BATCH // 8, 2, 8, 128), jnp.float32),
    scratch_types=[
        pltpu.VMEM((NSTEP, CHUNK), jnp.int32),
        pltpu.VMEM((NBUF, CHUNK, ATTR_DIM), jnp.float32),
        pltpu.SemaphoreType.DMA,
        pltpu.SemaphoreType.DMA((NBUF,)),
        pltpu.SemaphoreType.DMA((NBUF,)),
    ],
)
def _co_embed(idx, wall, out, idx_v, rows, isem, gsem, osem):
    wid = lax.axis_index("s") * NC + lax.axis_index("c")
    base = wid * BW

    def idx_src(k):
        f = k // NCHUNK
        c = lax.rem(k, NCHUNK) if not isinstance(k, int) else k % NCHUNK
        return idx.at[pl.ds(f * BATCH + base + c * CHUNK, CHUNK)]

    def stage(k, _):
        pltpu.async_copy(idx_src(k), idx_v.at[k], isem)
        return _

    def stage_wait(k, _):
        pltpu.make_async_copy(idx_src(k), idx_v.at[k], isem).wait()
        return _

    lax.fori_loop(0, NSTEP, stage, 0)

    def slot(k):
        return k % NBUF if isinstance(k, int) else lax.rem(k, NBUF)

    def gather(k):
        stage_wait(k, 0)
        pltpu.async_copy(wall.at[idx_v.at[k]], rows.at[slot(k)],
                         gsem.at[slot(k)])

    def gather_wait(k):
        pltpu.make_async_copy(wall.at[idx_v.at[k]], rows.at[slot(k)],
                              gsem.at[slot(k)]).wait()

    def wb_dst(k, t):
        f = k // NCHUNK
        c = lax.rem(k, NCHUNK) if not isinstance(k, int) else k % NCHUNK
        tile0 = (base + c * CHUNK) // 8
        return out.at[tile0 + t, f // 2, :,
                      pl.ds((f % 2) * ATTR_DIM, ATTR_DIM)]

    def writeback(k):
        def body(t, _):
            pltpu.async_copy(rows.at[slot(k), pl.ds(t * 8, 8)],
                             wb_dst(k, t), osem.at[slot(k)])
            return _
        lax.fori_loop(0, CHUNK // 8, body, 0)

    def writeback_wait(k):
        def body(t, _):
            pltpu.make_async_copy(rows.at[slot(k), pl.ds(t * 8, 8)],
                                  wb_dst(k, t), osem.at[slot(k)]).wait()
            return _
        lax.fori_loop(0, CHUNK // 8, body, 0)

    def pipe(k, _):
        @pl.when(k >= NBUF)
        def _w():
            writeback_wait(k - NBUF)
        gather(k)

        @pl.when(k >= NBUF - 1)
        def _g():
            gather_wait(k - (NBUF - 1))
            writeback(k - (NBUF - 1))
        return _

    lax.fori_loop(0, NSTEP, pipe, 0)
    for j in range(NSTEP - NBUF + 1, NSTEP):
        gather_wait(j)
        writeback(j)
    for j in range(NSTEP - NBUF, NSTEP):
        writeback_wait(j)


def kernel(inputs, W0, W1, W2, W3):
    wall = jnp.concatenate([W0, W1, W2, W3], axis=0)
    shifted = inputs + jnp.arange(NUM_FIELDS, dtype=inputs.dtype) * VOCAB
    out4 = _co_embed(shifted.T.reshape(-1), wall)
    return out4.transpose(0, 2, 1, 3).reshape(BATCH, NUM_FIELDS * ATTR_DIM)
